# baseline (speedup 1.0000x reference)
import functools

import jax
import jax.numpy as jnp
from jax import lax
from jax.experimental import pallas as pl
from jax.experimental.pallas import tpu as pltpu
from jax.experimental.pallas import tpu_sc as plsc

NC = 2
NS = 16
NW = NC * NS
E = 320000
CH = 80
EPT = E // NW
NCH = EPT // CH
PADS = {10000: 10240, 5000: 5120, 2500: 2560, 1250: 1280}
BI = 512

_mesh = lambda: plsc.VectorSubcoreMesh(
    core_axis_name="c", subcore_axis_name="s", num_cores=NC, num_subcores=NS)


def _f32(*shape):
    return jax.ShapeDtypeStruct(shape, jnp.float32)


def _i32(*shape):
    return jax.ShapeDtypeStruct(shape, jnp.int32)


def _tc_matmul(x, w):
    def body(x_ref, w_ref, o_ref):
        o_ref[...] = jnp.dot(x_ref[...], w_ref[...],
                             preferred_element_type=jnp.float32)
    return pl.pallas_call(
        body, out_shape=_f32(x.shape[0], w.shape[1]))(x, w)


def _tc_gcn_fin(degT, h0):
    Np = h0.shape[0]

    def body(d_ref, h_ref, g_ref, dsq_ref, idg_ref):
        deg = d_ref[:, 0:1] + d_ref[:, 1:2] + 1.0
        dsq = lax.rsqrt(deg)
        g_ref[...] = h_ref[...] * dsq
        dsq_ref[...] = dsq
        idg_ref[...] = 1.0 / deg

    grid = (Np // BI,)
    return pl.pallas_call(
        body,
        grid=grid,
        in_specs=[pl.BlockSpec((BI, 2), lambda b: (b, 0)),
                  pl.BlockSpec((BI, 128), lambda b: (b, 0))],
        out_specs=[pl.BlockSpec((BI, 128), lambda b: (b, 0)),
                   pl.BlockSpec((BI, 1), lambda b: (b, 0)),
                   pl.BlockSpec((BI, 1), lambda b: (b, 0))],
        out_shape=[_f32(Np, 128), _f32(Np, 1), _f32(Np, 1)],
    )(degT, h0)


def _score_block(out, p_ref, pid, n_valid):
    p = p_ref[...]
    pn = jnp.sqrt(jnp.sum(p * p))
    s = jnp.tanh(jnp.dot(out, p, preferred_element_type=jnp.float32)
                 / (pn + 1e-16))
    rows = pid * BI + lax.broadcasted_iota(jnp.int32, (BI, 1), 0)
    return jnp.where(rows < n_valid, s, -2.0)


def _tc_gcn_combine(S0, S1, h0, dsq, ideg, b, p, n_valid):
    Np = h0.shape[0]

    def body(s0_ref, s1_ref, h_ref, dsq_ref, idg_ref, b_ref, p_ref,
             xs_ref, s_ref):
        agg = (dsq_ref[...] * (s0_ref[...] + s1_ref[...])
               + h_ref[...] * idg_ref[...] + b_ref[...])
        s = _score_block(agg, p_ref, pl.program_id(0), n_valid)
        s_ref[...] = s
        xs_ref[...] = agg * s

    grid = (Np // BI,)
    blk = pl.BlockSpec((BI, 128), lambda bk: (bk, 0))
    col = pl.BlockSpec((BI, 1), lambda bk: (bk, 0))
    return pl.pallas_call(
        body,
        grid=grid,
        in_specs=[blk, blk, blk, col, col,
                  pl.BlockSpec((1, 128), lambda bk: (0, 0)),
                  pl.BlockSpec((128, 1), lambda bk: (0, 0))],
        out_specs=[blk, col],
        out_shape=[_f32(Np, 128), _f32(Np, 1)],
    )(S0, S1, h0, dsq, ideg, b, p)


def _tc_rank(s_col, s_row):
    Np = s_col.shape[0]

    def body(sc_ref, sr_ref, o_ref):
        pid = pl.program_id(0)
        scol = sc_ref[...]
        i_ids = pid * BI + lax.broadcasted_iota(jnp.int32, (BI, 1), 0)

        def step(j, acc):
            srow = sr_ref[:, pl.ds(j * BI, BI)]
            j_ids = j * BI + lax.broadcasted_iota(jnp.int32, (1, BI), 1)
            gt = (srow > scol).astype(jnp.int32)
            eqlt = ((srow == scol) & (j_ids < i_ids)).astype(jnp.int32)
            return acc + jnp.sum(gt + eqlt, axis=1, keepdims=True)

        o_ref[...] = lax.fori_loop(0, Np // BI, step,
                                   jnp.zeros((BI, 1), jnp.int32))

    return pl.pallas_call(
        body,
        grid=(Np // BI,),
        in_specs=[pl.BlockSpec((BI, 1), lambda b: (b, 0)),
                  pl.BlockSpec((1, Np), lambda b: (0, 0))],
        out_specs=pl.BlockSpec((BI, 1), lambda b: (b, 0)),
        out_shape=_i32(Np, 1),
    )(s_col, s_row)


def _tc_gat_pre(x, W, a_s, a_d):
    Np = x.shape[0]

    def body(x_ref, w_ref, as_ref, ad_ref,
             h_ref, als_ref, ald_ref, exs_ref, u_ref, v_ref, k_ref):
        h = jnp.dot(x_ref[...], w_ref[...], preferred_element_type=jnp.float32)
        h_ref[...] = h
        als = jnp.dot(h, as_ref[...], preferred_element_type=jnp.float32)
        ald = jnp.dot(h, ad_ref[...], preferred_element_type=jnp.float32)
        als_ref[...] = als
        ald_ref[...] = ald
        Ms = jnp.max(als)
        m = Ms + jnp.max(ald)
        c = jnp.maximum(m, 0.2 * m)
        ct = 0.2 * Ms
        lanes = lax.broadcasted_iota(jnp.int32, (1, 128), 1)
        k_row = jnp.where(lanes == 0, c,
                          jnp.where(lanes == 1, Ms,
                                    jnp.where(lanes == 2, ct, 0.0)))
        k_ref[...] = k_row
        t = als + ald
        e = jnp.maximum(t, 0.2 * t)
        exs_ref[...] = jnp.exp(e - c)
        u_ref[...] = jnp.exp(als - Ms) * h
        v_ref[...] = jnp.exp(0.2 * als - ct) * h

    return pl.pallas_call(
        body,
        out_shape=[_f32(Np, 128), _f32(Np, 1), _f32(Np, 1), _f32(Np, 1),
                   _f32(Np, 128), _f32(Np, 128), _f32(1, 128)],
    )(x, W, a_s, a_d)


def _tc_gat_post(Rp0, Rp1, Rm0, Rm1, h, exs, ald, dpT, dmT, consts,
                 b, p, n_valid):
    Np = h.shape[0]

    def body(rp0_ref, rp1_ref, rm0_ref, rm1_ref, h_ref, exs_ref, ald_ref,
             dp_ref, dm_ref, k_ref, b_ref, p_ref, xs_ref, s_ref):
        krow = k_ref[...]
        c = krow[0, 0]
        Ms = krow[0, 1]
        ct = krow[0, 2]
        exs = exs_ref[...]
        ald_v = ald_ref[...]
        ea = jnp.exp(ald_v - (c - Ms))
        eb = jnp.exp(0.2 * ald_v - (c - ct))
        num = (ea * (rp0_ref[...] + rp1_ref[...])
               + eb * (rm0_ref[...] + rm1_ref[...]) + exs * h_ref[...])
        den = (ea * (dp_ref[:, 0:1] + dp_ref[:, 1:2])
               + eb * (dm_ref[:, 0:1] + dm_ref[:, 1:2]) + exs)
        out = num / (den + 1e-16) + b_ref[...]
        s = _score_block(out, p_ref, pl.program_id(0), n_valid)
        s_ref[...] = s
        xs_ref[...] = out * s

    grid = (Np // BI,)
    blk = pl.BlockSpec((BI, 128), lambda bk: (bk, 0))
    col = pl.BlockSpec((BI, 1), lambda bk: (bk, 0))
    two = pl.BlockSpec((BI, 2), lambda bk: (bk, 0))
    return pl.pallas_call(
        body,
        grid=grid,
        in_specs=[blk, blk, blk, blk, blk, col, col, two, two,
                  pl.BlockSpec((1, 128), lambda bk: (0, 0)),
                  pl.BlockSpec((1, 128), lambda bk: (0, 0)),
                  pl.BlockSpec((128, 1), lambda bk: (0, 0))],
        out_specs=[blk, col],
        out_shape=[_f32(Np, 128), _f32(Np, 1)],
    )(Rp0, Rp1, Rm0, Rm1, h, exs, ald, dpT, dmT, consts, b, p)


def _tc_head(h3, eps, W_mu, b_mu, W_lv, b_lv, W_lat_p, b_lat_p,
             Wd2_p, bd2_p, Wd1_p, bd1_p, Wd0_p, bd0_p):
    Np = h3.shape[0]

    def body(h_ref, e_ref, wmu_ref, bmu_ref, wlv_ref, blv_ref,
             wlat_ref, blat_ref, w2_ref, b2_ref, w1_ref, b1_ref,
             w0_ref, b0_ref, z_ref, mu_ref, lv_ref):
        h = h_ref[...]
        mu = jnp.dot(h, wmu_ref[...], preferred_element_type=jnp.float32) \
            + bmu_ref[...]
        lv = jnp.dot(h, wlv_ref[...], preferred_element_type=jnp.float32) \
            + blv_ref[...]
        mu_ref[...] = mu
        lv_ref[...] = lv
        z = mu + e_ref[...] * jnp.exp(0.5 * lv)
        z = jnp.dot(z, wlat_ref[...], preferred_element_type=jnp.float32) \
            + blat_ref[...]
        for w_r, b_r in ((w2_ref, b2_ref), (w1_ref, b1_ref), (w0_ref, b0_ref)):
            z = jnp.dot(z, w_r[...], preferred_element_type=jnp.float32) \
                + b_r[...]
        z_ref[...] = z

    return pl.pallas_call(
        body,
        out_shape=[_f32(Np, 128), _f32(Np, 32), _f32(Np, 32)],
    )(h3, eps, W_mu, b_mu, W_lv, b_lv, W_lat_p, b_lat_p,
      Wd2_p, bd2_p, Wd1_p, bd1_p, Wd0_p, bd0_p)


def _sc_deg(dst, Np):
    RT = Np // NS

    @functools.partial(
        pl.kernel,
        out_type=_f32(NC * Np),
        mesh=_mesh(),
        compiler_params=pltpu.CompilerParams(needs_layout_passes=False),
        scratch_types=[
            pltpu.VMEM((CH,), jnp.int32),
            pltpu.VMEM((CH,), jnp.float32),
            pltpu.VMEM((CH,), jnp.float32),
            pltpu.VMEM_SHARED((Np,), jnp.float32),
        ],
    )
    def k(dst_hbm, out_hbm, dbuf, ones, zb, shdeg):
        c = lax.axis_index("c")
        s = lax.axis_index("s")
        for g in range(CH // 16):
            ones[pl.ds(g * 16, 16)] = jnp.ones((16,), jnp.float32)
            zb[pl.ds(g * 16, 16)] = jnp.zeros((16,), jnp.float32)
        for j in range(RT // CH):
            pltpu.sync_copy(zb, shdeg.at[pl.ds(s * RT + j * CH, CH)])
        plsc.subcore_barrier()
        ebase = c * (E // NC) + s * EPT

        def step(i, carry):
            off = ebase + i * CH
            pltpu.sync_copy(dst_hbm.at[pl.ds(off, CH)], dbuf)
            pltpu.sync_copy(ones, shdeg.at[dbuf], add=True)
            return carry

        lax.fori_loop(0, NCH, step, 0)
        plsc.subcore_barrier()
        for j in range(RT // CH):
            pltpu.sync_copy(shdeg.at[pl.ds(s * RT + j * CH, CH)], zb)
            pltpu.sync_copy(zb, out_hbm.at[pl.ds(c * Np + s * RT + j * CH,
                                                 CH)])

    return k(dst).reshape(NC, Np)


def _zero_rows(rowbuf):
    for r in range(CH):
        for w in range(8):
            rowbuf[r, pl.ds(w * 16, 16)] = jnp.zeros((16,), jnp.float32)


def _sc_scatter_rows(g, src, dst):
    Np = g.shape[0]
    RT = Np // NS

    @functools.partial(
        pl.kernel,
        out_type=_f32(NC, Np, 128),
        mesh=_mesh(),
        compiler_params=pltpu.CompilerParams(needs_layout_passes=False),
        scratch_types=[
            pltpu.VMEM((CH,), jnp.int32),
            pltpu.VMEM((CH,), jnp.int32),
            pltpu.VMEM((CH, 128), jnp.float32),
            pltpu.SemaphoreType.DMA,
            pltpu.VMEM_SHARED((Np, 128), jnp.float32),
        ],
    )
    def k(g_hbm, src_hbm, dst_hbm, out_hbm, sbuf, dbuf, rowbuf, sem,
          shR):
        c = lax.axis_index("c")
        s = lax.axis_index("s")
        _zero_rows(rowbuf)
        for j in range(RT // CH):
            pltpu.sync_copy(rowbuf, shR.at[pl.ds(s * RT + j * CH, CH), :])
        plsc.subcore_barrier()
        ebase = c * (E // NC) + s * EPT

        def step(i, carry):
            off = ebase + i * CH
            pltpu.sync_copy(src_hbm.at[pl.ds(off, CH)], sbuf)
            pltpu.async_copy(g_hbm.at[sbuf], rowbuf, sem).wait()
            pltpu.sync_copy(dst_hbm.at[pl.ds(off, CH)], dbuf)
            pltpu.sync_copy(rowbuf, shR.at[dbuf], add=True)
            return carry

        lax.fori_loop(0, NCH, step, 0)
        plsc.subcore_barrier()
        for j in range(RT // CH):
            pltpu.sync_copy(shR.at[pl.ds(s * RT + j * CH, CH), :], rowbuf)
            pltpu.sync_copy(rowbuf,
                            out_hbm.at[c, pl.ds(s * RT + j * CH, CH), :])

    return k(g, src, dst)


def _sc_gat_cls(als, ald, c16, src, dst, emask, Np):
    ZROW = Np
    RT = Np // NS

    @functools.partial(
        pl.kernel,
        out_type=(_f32(NC * Np), _f32(NC * Np), _i32(E), _i32(E)),
        mesh=_mesh(),
        compiler_params=pltpu.CompilerParams(needs_layout_passes=False),
        scratch_types=[
            pltpu.VMEM((Np,), jnp.float32),
            pltpu.VMEM((Np,), jnp.float32),
            pltpu.VMEM((16,), jnp.float32),
            pltpu.VMEM((CH,), jnp.int32),
            pltpu.VMEM((CH,), jnp.int32),
            pltpu.VMEM((CH,), jnp.float32),
            pltpu.VMEM((CH,), jnp.int32),
            pltpu.VMEM((CH,), jnp.int32),
            pltpu.VMEM((CH,), jnp.float32),
            pltpu.VMEM((CH,), jnp.float32),
            pltpu.VMEM_SHARED((Np,), jnp.float32),
            pltpu.VMEM_SHARED((Np,), jnp.float32),
        ],
    )
    def k(als_hbm, ald_hbm, c_hbm, src_hbm, dst_hbm, em_hbm,
          denp_hbm, denm_hbm, sp_hbm, sm_hbm,
          alsb, aldb, cb, sbuf, dbuf, mbuf, spb, smb, enp, enm, shdp, shdm):
        c_ = lax.axis_index("c")
        s = lax.axis_index("s")
        for g in range(CH // 16):
            enp[pl.ds(g * 16, 16)] = jnp.zeros((16,), jnp.float32)
        for j in range(RT // CH):
            o = s * RT + j * CH
            pltpu.sync_copy(enp, shdp.at[pl.ds(o, CH)])
            pltpu.sync_copy(enp, shdm.at[pl.ds(o, CH)])
        pltpu.sync_copy(als_hbm, alsb)
        pltpu.sync_copy(ald_hbm, aldb)
        pltpu.sync_copy(c_hbm, cb)
        plsc.subcore_barrier()
        kvec = cb[pl.ds(0, 16)]
        Ms = kvec[1]
        ct = kvec[2]
        ebase = c_ * (E // NC) + s * EPT

        def step(i, carry):
            off = ebase + i * CH
            pltpu.sync_copy(src_hbm.at[pl.ds(off, CH)], sbuf)
            pltpu.sync_copy(dst_hbm.at[pl.ds(off, CH)], dbuf)
            pltpu.sync_copy(em_hbm.at[pl.ds(off, CH)], mbuf)
            for g in range(CH // 16):
                sl = pl.ds(g * 16, 16)
                si = sbuf[sl]
                di = dbuf[sl]
                av = plsc.load_gather(alsb, [si])
                dv = plsc.load_gather(aldb, [di])
                t = av + dv
                live = mbuf[sl] > 0.0
                kp = (t >= 0.0) & live
                km = (t < 0.0) & live
                spb[sl] = jnp.where(kp, si, ZROW)
                smb[sl] = jnp.where(km, si, ZROW)
                enp[sl] = jnp.where(kp, jnp.exp(av - Ms), 0.0)
                enm[sl] = jnp.where(km, jnp.exp(0.2 * av - ct), 0.0)
            pltpu.sync_copy(enp, shdp.at[dbuf], add=True)
            pltpu.sync_copy(enm, shdm.at[dbuf], add=True)
            pltpu.sync_copy(spb, sp_hbm.at[pl.ds(off, CH)])
            pltpu.sync_copy(smb, sm_hbm.at[pl.ds(off, CH)])
            return carry

        lax.fori_loop(0, NCH, step, 0)
        plsc.subcore_barrier()
        for j in range(RT // CH):
            o = s * RT + j * CH
            pltpu.sync_copy(shdp.at[pl.ds(o, CH)], enp)
            pltpu.sync_copy(enp, denp_hbm.at[pl.ds(c_ * Np + o, CH)])
            pltpu.sync_copy(shdm.at[pl.ds(o, CH)], enm)
            pltpu.sync_copy(enm, denm_hbm.at[pl.ds(c_ * Np + o, CH)])

    dp, dm, sp, sm = k(als, ald, c16, src, dst, emask)
    return dp.reshape(NC, Np), dm.reshape(NC, Np), sp, sm


def _sc_gat_rows(u, v, sp, sm, dst, Np):
    RT = Np // NS

    @functools.partial(
        pl.kernel,
        out_type=(_f32(NC, Np, 128), _f32(NC, Np, 128)),
        mesh=_mesh(),
        compiler_params=pltpu.CompilerParams(needs_layout_passes=False),
        scratch_types=[
            pltpu.VMEM((CH,), jnp.int32),
            pltpu.VMEM((CH,), jnp.int32),
            pltpu.VMEM((CH,), jnp.int32),
            pltpu.VMEM((CH, 128), jnp.float32),
            pltpu.VMEM((CH, 128), jnp.float32),
            pltpu.SemaphoreType.DMA,
            pltpu.SemaphoreType.DMA,
            pltpu.VMEM_SHARED((Np, 128), jnp.float32),
            pltpu.VMEM_SHARED((Np, 128), jnp.float32),
        ],
    )
    def k(u_hbm, v_hbm, sp_hbm, sm_hbm, dst_hbm, Rp_hbm, Rm_hbm,
          spb, smb, dbuf, rowp, rowm, sem1, sem2, shRp, shRm):
        c_ = lax.axis_index("c")
        s = lax.axis_index("s")
        _zero_rows(rowp)
        for j in range(RT // CH):
            o = s * RT + j * CH
            pltpu.sync_copy(rowp, shRp.at[pl.ds(o, CH), :])
            pltpu.sync_copy(rowp, shRm.at[pl.ds(o, CH), :])
        plsc.subcore_barrier()
        ebase = c_ * (E // NC) + s * EPT

        def step(i, carry):
            off = ebase + i * CH
            pltpu.sync_copy(sp_hbm.at[pl.ds(off, CH)], spb)
            gp = pltpu.async_copy(u_hbm.at[spb], rowp, sem1)
            pltpu.sync_copy(sm_hbm.at[pl.ds(off, CH)], smb)
            gm = pltpu.async_copy(v_hbm.at[smb], rowm, sem2)
            pltpu.sync_copy(dst_hbm.at[pl.ds(off, CH)], dbuf)
            gp.wait()
            pltpu.sync_copy(rowp, shRp.at[dbuf], add=True)
            gm.wait()
            pltpu.sync_copy(rowm, shRm.at[dbuf], add=True)
            return carry

        lax.fori_loop(0, NCH, step, 0)
        plsc.subcore_barrier()
        for j in range(RT // CH):
            o = s * RT + j * CH
            pltpu.sync_copy(shRp.at[pl.ds(o, CH), :], rowp)
            pltpu.sync_copy(rowp, Rp_hbm.at[c_, pl.ds(o, CH), :])
            pltpu.sync_copy(shRm.at[pl.ds(o, CH), :], rowm)
            pltpu.sync_copy(rowm, Rm_hbm.at[c_, pl.ds(o, CH), :])

    return k(u, v, sp, sm, dst)


def _sc_gat(u, v, als, ald, c16, src, dst, emask):
    Np = als.shape[0]
    dp, dm, sp, sm = _sc_gat_cls(als, ald, c16, src, dst, emask, Np)
    Rp, Rm = _sc_gat_rows(u, v, sp, sm, dst, Np)
    return dp, dm, Rp, Rm


def _sc_pool(xs, rank, kk, npad_next=0, edges=None):
    Np = xs.shape[0]
    RT32 = Np // NW
    RCH = RT32 // CH if RT32 >= CH else 0

    with_edges = edges is not None
    out_type = [_f32(Np, 128)]
    if with_edges:
        out_type += [_i32(E), _i32(E), _f32(E)]

    scratch = [
        pltpu.VMEM((CH,), jnp.int32),
        pltpu.VMEM((CH, 128), jnp.float32),
        pltpu.VMEM((Np,), jnp.int32),
        pltpu.VMEM((CH,), jnp.int32),
        pltpu.VMEM((CH,), jnp.float32),
    ]

    def body(*refs):
        if with_edges:
            (xs_hbm, rk_hbm, src_hbm, dst_hbm, em_hbm,
             out_hbm, ns_hbm, nd_hbm, nm_hbm,
             rbuf, rowbuf, rktab, dbuf, mbuf) = refs
        else:
            (xs_hbm, rk_hbm, out_hbm,
             rbuf, rowbuf, rktab, dbuf, mbuf) = refs
        c = lax.axis_index("c")
        s = lax.axis_index("s")
        wid = c * NS + s
        rbase = wid * RT32

        if RCH:
            def rstep(i, carry):
                off = rbase + i * CH
                pltpu.sync_copy(xs_hbm.at[pl.ds(off, CH), :], rowbuf)
                pltpu.sync_copy(rk_hbm.at[pl.ds(off, CH)], rbuf)
                pltpu.sync_copy(rowbuf, out_hbm.at[rbuf])
                return carry
            lax.fori_loop(0, RCH, rstep, 0)

        if with_edges:
            pltpu.sync_copy(rk_hbm, rktab)
            ebase = c * (E // NC) + s * EPT
            kv = jnp.int32(kk)
            GPAD = jnp.int32(npad_next - kk)

            def estep(i, carry):
                off = ebase + i * CH
                pltpu.sync_copy(src_hbm.at[pl.ds(off, CH)], rbuf)
                pltpu.sync_copy(dst_hbm.at[pl.ds(off, CH)], dbuf)
                pltpu.sync_copy(em_hbm.at[pl.ds(off, CH)], mbuf)
                for g in range(CH // 16):
                    si = rbuf[pl.ds(g * 16, 16)]
                    di = dbuf[pl.ds(g * 16, 16)]
                    rs = plsc.load_gather(rktab, [si])
                    rd = plsc.load_gather(rktab, [di])
                    ks = rs < kv
                    kd = rd < kv
                    rbuf[pl.ds(g * 16, 16)] = jnp.where(ks, rs, 0)
                    dbuf[pl.ds(g * 16, 16)] = jnp.where(
                        kd, rd, kv + jnp.remainder(rd, GPAD))
                    mbuf[pl.ds(g * 16, 16)] = jnp.where(
                        ks & kd, mbuf[pl.ds(g * 16, 16)], 0.0)
                pltpu.sync_copy(rbuf, ns_hbm.at[pl.ds(off, CH)])
                pltpu.sync_copy(dbuf, nd_hbm.at[pl.ds(off, CH)])
                pltpu.sync_copy(mbuf, nm_hbm.at[pl.ds(off, CH)])
                return carry

            lax.fori_loop(0, NCH, estep, 0)

    kfun = functools.partial(
        pl.kernel, out_type=tuple(out_type), mesh=_mesh(),
        compiler_params=pltpu.CompilerParams(needs_layout_passes=False),
        scratch_types=scratch)(body)
    if with_edges:
        return kfun(xs, rank, *edges)
    return kfun(xs, rank)


def _pad_rows(a, np_):
    return jnp.pad(a, ((0, np_ - a.shape[0]),) + ((0, 0),) * (a.ndim - 1))


def kernel(x, edge_index, W_enc0, b_enc0, p0, W_gat1, b_gat1, a_src1, a_dst1,
           p1, W_gat2, b_gat2, a_src2, a_dst2, p2, W_mu, b_mu, W_lv, b_lv,
           W_lat, b_lat, W_d2, b_d2, W_d1, b_d1, W_d0, b_d0):
    src = edge_index[0]
    dst = edge_index[1]
    N0, Np0 = 10000, PADS[10000]
    k0, k1, k2 = 5000, 2500, 1250

    x_pad = jnp.pad(x, ((0, Np0 - N0), (0, 5)))
    W8 = jnp.pad(W_enc0, ((0, 5), (0, 0)))
    h0 = _tc_matmul(x_pad, W8)
    degparts = _sc_deg(dst, Np0)
    g, dsq, ideg = _tc_gcn_fin(degparts.T, h0)
    Sparts = _sc_scatter_rows(g, src, dst)
    xs0, s0 = _tc_gcn_combine(Sparts[0], Sparts[1], h0, dsq, ideg,
                              b_enc0.reshape(1, 128), p0.reshape(128, 1), N0)
    rank0 = _tc_rank(s0, s0.reshape(1, Np0))
    em0 = jnp.ones((E,), jnp.float32)
    sorted0, src1, dst1, em1 = _sc_pool(
        xs0, rank0.reshape(Np0), k0, PADS[k0], (src, dst, em0))

    def gat_level(x_lv, W, a_s, a_d, b, p, s_e, d_e, m_e, n_valid):
        Np = x_lv.shape[0]
        h, als, ald, exs, u, vtab, consts = _tc_gat_pre(
            x_lv, W, a_s.reshape(128, 1), a_d.reshape(128, 1))
        c16 = consts[0, :16].reshape(16)
        u_pad = jnp.pad(u, ((0, 8), (0, 0)))
        v_pad = jnp.pad(vtab, ((0, 8), (0, 0)))
        dp, dm, Rp, Rm = _sc_gat(u_pad, v_pad, als.reshape(Np),
                                 ald.reshape(Np), c16, s_e, d_e, m_e)
        xs, s = _tc_gat_post(Rp[0], Rp[1], Rm[0], Rm[1], h, exs, ald,
                             dp.T, dm.T, consts, b.reshape(1, 128),
                             p.reshape(128, 1), n_valid)
        rank = _tc_rank(s, s.reshape(1, Np))
        return xs, rank.reshape(Np)

    x1 = sorted0[:PADS[k0]]
    xs1, rank1 = gat_level(x1, W_gat1, a_src1, a_dst1, b_gat1, p1,
                           src1, dst1, em1, k0)
    sorted1, src2, dst2, em2 = _sc_pool(xs1, rank1, k1, PADS[k1],
                                        (src1, dst1, em1))

    x2 = sorted1[:PADS[k1]]
    xs2, rank2 = gat_level(x2, W_gat2, a_src2, a_dst2, b_gat2, p2,
                           src2, dst2, em2, k1)
    (sorted2,) = _sc_pool(xs2, rank2, k2)

    h3 = sorted2[:PADS[k2]]
    eps = jax.random.normal(jax.random.key(42), (k2, 32), dtype=jnp.float32)
    eps = _pad_rows(eps, PADS[k2])
    pad33 = lambda w: jnp.pad(w, ((0, 125), (0, 125)))
    padb = lambda b: jnp.pad(b, (0, 125)).reshape(1, 128)
    W_lat_p = jnp.pad(W_lat, ((0, 0), (0, 125)))
    b_lat_p = padb(b_lat)
    z, mu, lv = _tc_head(
        h3, eps, W_mu, b_mu.reshape(1, 32), W_lv, b_lv.reshape(1, 32),
        W_lat_p, b_lat_p, pad33(W_d2), padb(b_d2), pad33(W_d1), padb(b_d1),
        pad33(W_d0), padb(b_d0))
    return (z[:k2, :3], mu[:k2], lv[:k2])


# speedup vs baseline: 9.7267x; 9.7267x over previous
import functools

import jax
import jax.numpy as jnp
from jax import lax
from jax.experimental import pallas as pl
from jax.experimental.pallas import tpu as pltpu
from jax.experimental.pallas import tpu_sc as plsc

NC = 2
NS = 16
NW = NC * NS
E = 320000
CH = 80
EPT = E // NW
NCH = EPT // CH
PADS = {10000: 10240, 5000: 5120, 2500: 2560, 1250: 1280}
BI = 512

_mesh = lambda: plsc.VectorSubcoreMesh(
    core_axis_name="c", subcore_axis_name="s", num_cores=NC, num_subcores=NS)


def _f32(*shape):
    return jax.ShapeDtypeStruct(shape, jnp.float32)


def _i32(*shape):
    return jax.ShapeDtypeStruct(shape, jnp.int32)


def _tc_matmul(x, w):
    def body(x_ref, w_ref, o_ref):
        o_ref[...] = jnp.dot(x_ref[...], w_ref[...],
                             preferred_element_type=jnp.float32)
    return pl.pallas_call(
        body, out_shape=_f32(x.shape[0], w.shape[1]))(x, w)


def _tc_gcn_fin(degT, h0):
    Np = h0.shape[0]

    def body(d_ref, h_ref, g_ref, dsq_ref, idg_ref):
        deg = d_ref[:, 0:1] + d_ref[:, 1:2] + 1.0
        dsq = lax.rsqrt(deg)
        g_ref[...] = h_ref[...] * dsq
        dsq_ref[...] = dsq
        idg_ref[...] = 1.0 / deg

    grid = (Np // BI,)
    return pl.pallas_call(
        body,
        grid=grid,
        in_specs=[pl.BlockSpec((BI, 2), lambda b: (b, 0)),
                  pl.BlockSpec((BI, 128), lambda b: (b, 0))],
        out_specs=[pl.BlockSpec((BI, 128), lambda b: (b, 0)),
                   pl.BlockSpec((BI, 1), lambda b: (b, 0)),
                   pl.BlockSpec((BI, 1), lambda b: (b, 0))],
        out_shape=[_f32(Np, 128), _f32(Np, 1), _f32(Np, 1)],
    )(degT, h0)


def _score_block(out, p_ref, pid, n_valid):
    p = p_ref[...]
    pn = jnp.sqrt(jnp.sum(p * p))
    s = jnp.tanh(jnp.dot(out, p, preferred_element_type=jnp.float32)
                 / (pn + 1e-16))
    rows = pid * BI + lax.broadcasted_iota(jnp.int32, (BI, 1), 0)
    return jnp.where(rows < n_valid, s, -2.0)


def _tc_gcn_combine(S0, S1, h0, dsq, ideg, b, p, n_valid):
    Np = h0.shape[0]

    def body(s0_ref, s1_ref, h_ref, dsq_ref, idg_ref, b_ref, p_ref,
             xs_ref, s_ref):
        agg = (dsq_ref[...] * (s0_ref[...] + s1_ref[...])
               + h_ref[...] * idg_ref[...] + b_ref[...])
        s = _score_block(agg, p_ref, pl.program_id(0), n_valid)
        s_ref[...] = s
        xs_ref[...] = agg * s

    grid = (Np // BI,)
    blk = pl.BlockSpec((BI, 128), lambda bk: (bk, 0))
    col = pl.BlockSpec((BI, 1), lambda bk: (bk, 0))
    return pl.pallas_call(
        body,
        grid=grid,
        in_specs=[blk, blk, blk, col, col,
                  pl.BlockSpec((1, 128), lambda bk: (0, 0)),
                  pl.BlockSpec((128, 1), lambda bk: (0, 0))],
        out_specs=[blk, col],
        out_shape=[_f32(Np, 128), _f32(Np, 1)],
    )(S0, S1, h0, dsq, ideg, b, p)


def _tc_rank(s_col, s_row):
    Np = s_col.shape[0]

    def body(sc_ref, sr_ref, o_ref):
        pid = pl.program_id(0)
        scol = sc_ref[...]
        i_ids = pid * BI + lax.broadcasted_iota(jnp.int32, (BI, 1), 0)

        def step(j, acc):
            srow = sr_ref[:, pl.ds(j * BI, BI)]
            j_ids = j * BI + lax.broadcasted_iota(jnp.int32, (1, BI), 1)
            gt = (srow > scol).astype(jnp.int32)
            eqlt = ((srow == scol) & (j_ids < i_ids)).astype(jnp.int32)
            return acc + jnp.sum(gt + eqlt, axis=1, keepdims=True)

        o_ref[...] = lax.fori_loop(0, Np // BI, step,
                                   jnp.zeros((BI, 1), jnp.int32))

    return pl.pallas_call(
        body,
        grid=(Np // BI,),
        in_specs=[pl.BlockSpec((BI, 1), lambda b: (b, 0)),
                  pl.BlockSpec((1, Np), lambda b: (0, 0))],
        out_specs=pl.BlockSpec((BI, 1), lambda b: (b, 0)),
        out_shape=_i32(Np, 1),
    )(s_col, s_row)


def _tc_gat_pre(x, W, a_s, a_d):
    Np = x.shape[0]

    def body(x_ref, w_ref, as_ref, ad_ref,
             h_ref, als_ref, ald_ref, exs_ref, u_ref, v_ref, k_ref):
        h = jnp.dot(x_ref[...], w_ref[...], preferred_element_type=jnp.float32)
        h_ref[...] = h
        als = jnp.dot(h, as_ref[...], preferred_element_type=jnp.float32)
        ald = jnp.dot(h, ad_ref[...], preferred_element_type=jnp.float32)
        als_ref[...] = als
        ald_ref[...] = ald
        Ms = jnp.max(als)
        m = Ms + jnp.max(ald)
        c = jnp.maximum(m, 0.2 * m)
        ct = 0.2 * Ms
        lanes = lax.broadcasted_iota(jnp.int32, (1, 128), 1)
        k_row = jnp.where(lanes == 0, c,
                          jnp.where(lanes == 1, Ms,
                                    jnp.where(lanes == 2, ct, 0.0)))
        k_ref[...] = k_row
        t = als + ald
        e = jnp.maximum(t, 0.2 * t)
        exs_ref[...] = jnp.exp(e - c)
        u_ref[...] = jnp.exp(als - Ms) * h
        v_ref[...] = jnp.exp(0.2 * als - ct) * h

    return pl.pallas_call(
        body,
        out_shape=[_f32(Np, 128), _f32(Np, 1), _f32(Np, 1), _f32(Np, 1),
                   _f32(Np, 128), _f32(Np, 128), _f32(1, 128)],
    )(x, W, a_s, a_d)


def _tc_gat_post(Rp0, Rp1, Rm0, Rm1, h, exs, ald, dpT, dmT, consts,
                 b, p, n_valid):
    Np = h.shape[0]

    def body(rp0_ref, rp1_ref, rm0_ref, rm1_ref, h_ref, exs_ref, ald_ref,
             dp_ref, dm_ref, k_ref, b_ref, p_ref, xs_ref, s_ref):
        krow = k_ref[...]
        c = krow[0, 0]
        Ms = krow[0, 1]
        ct = krow[0, 2]
        exs = exs_ref[...]
        ald_v = ald_ref[...]
        ea = jnp.exp(ald_v - (c - Ms))
        eb = jnp.exp(0.2 * ald_v - (c - ct))
        num = (ea * (rp0_ref[...] + rp1_ref[...])
               + eb * (rm0_ref[...] + rm1_ref[...]) + exs * h_ref[...])
        den = (ea * (dp_ref[:, 0:1] + dp_ref[:, 1:2])
               + eb * (dm_ref[:, 0:1] + dm_ref[:, 1:2]) + exs)
        out = num / (den + 1e-16) + b_ref[...]
        s = _score_block(out, p_ref, pl.program_id(0), n_valid)
        s_ref[...] = s
        xs_ref[...] = out * s

    grid = (Np // BI,)
    blk = pl.BlockSpec((BI, 128), lambda bk: (bk, 0))
    col = pl.BlockSpec((BI, 1), lambda bk: (bk, 0))
    two = pl.BlockSpec((BI, 2), lambda bk: (bk, 0))
    return pl.pallas_call(
        body,
        grid=grid,
        in_specs=[blk, blk, blk, blk, blk, col, col, two, two,
                  pl.BlockSpec((1, 128), lambda bk: (0, 0)),
                  pl.BlockSpec((1, 128), lambda bk: (0, 0)),
                  pl.BlockSpec((128, 1), lambda bk: (0, 0))],
        out_specs=[blk, col],
        out_shape=[_f32(Np, 128), _f32(Np, 1)],
    )(Rp0, Rp1, Rm0, Rm1, h, exs, ald, dpT, dmT, consts, b, p)


def _tc_head(h3, eps, W_mu, b_mu, W_lv, b_lv, W_lat_p, b_lat_p,
             Wd2_p, bd2_p, Wd1_p, bd1_p, Wd0_p, bd0_p):
    Np = h3.shape[0]

    def body(h_ref, e_ref, wmu_ref, bmu_ref, wlv_ref, blv_ref,
             wlat_ref, blat_ref, w2_ref, b2_ref, w1_ref, b1_ref,
             w0_ref, b0_ref, z_ref, mu_ref, lv_ref):
        h = h_ref[...]
        mu = jnp.dot(h, wmu_ref[...], preferred_element_type=jnp.float32) \
            + bmu_ref[...]
        lv = jnp.dot(h, wlv_ref[...], preferred_element_type=jnp.float32) \
            + blv_ref[...]
        mu_ref[...] = mu
        lv_ref[...] = lv
        z = mu + e_ref[...] * jnp.exp(0.5 * lv)
        z = jnp.dot(z, wlat_ref[...], preferred_element_type=jnp.float32) \
            + blat_ref[...]
        for w_r, b_r in ((w2_ref, b2_ref), (w1_ref, b1_ref), (w0_ref, b0_ref)):
            z = jnp.dot(z, w_r[...], preferred_element_type=jnp.float32) \
                + b_r[...]
        z_ref[...] = z

    return pl.pallas_call(
        body,
        out_shape=[_f32(Np, 128), _f32(Np, 32), _f32(Np, 32)],
    )(h3, eps, W_mu, b_mu, W_lv, b_lv, W_lat_p, b_lat_p,
      Wd2_p, bd2_p, Wd1_p, bd1_p, Wd0_p, bd0_p)


def _sc_deg(dst, Np):
    RT = Np // NS

    @functools.partial(
        pl.kernel,
        out_type=_f32(NC * Np),
        mesh=_mesh(),
        compiler_params=pltpu.CompilerParams(needs_layout_passes=False),
        scratch_types=[
            pltpu.VMEM((CH,), jnp.int32),
            pltpu.VMEM((CH,), jnp.float32),
            pltpu.VMEM((CH,), jnp.float32),
            pltpu.VMEM_SHARED((Np,), jnp.float32),
        ],
    )
    def k(dst_hbm, out_hbm, dbuf, ones, zb, shdeg):
        c = lax.axis_index("c")
        s = lax.axis_index("s")
        for g in range(CH // 16):
            ones[pl.ds(g * 16, 16)] = jnp.ones((16,), jnp.float32)
            zb[pl.ds(g * 16, 16)] = jnp.zeros((16,), jnp.float32)
        for j in range(RT // CH):
            pltpu.sync_copy(zb, shdeg.at[pl.ds(s * RT + j * CH, CH)])
        plsc.subcore_barrier()
        ebase = c * (E // NC) + s * EPT

        def step(i, carry):
            off = ebase + i * CH
            pltpu.sync_copy(dst_hbm.at[pl.ds(off, CH)], dbuf)
            pltpu.sync_copy(ones, shdeg.at[dbuf], add=True)
            return carry

        lax.fori_loop(0, NCH, step, 0)
        plsc.subcore_barrier()
        for j in range(RT // CH):
            pltpu.sync_copy(shdeg.at[pl.ds(s * RT + j * CH, CH)], zb)
            pltpu.sync_copy(zb, out_hbm.at[pl.ds(c * Np + s * RT + j * CH,
                                                 CH)])

    return k(dst).reshape(NC, Np)


def _zero_rows(rowbuf):
    for r in range(CH):
        for w in range(8):
            rowbuf[r, pl.ds(w * 16, 16)] = jnp.zeros((16,), jnp.float32)


def _sc_scatter_rows(g, src, dst):
    Np = g.shape[0]
    RT = Np // NS

    @functools.partial(
        pl.kernel,
        out_type=_f32(NC, Np, 128),
        mesh=_mesh(),
        compiler_params=pltpu.CompilerParams(needs_layout_passes=False),
        scratch_types=[
            pltpu.VMEM((CH,), jnp.int32),
            pltpu.VMEM((CH,), jnp.int32),
            pltpu.VMEM((CH, 128), jnp.float32),
            pltpu.SemaphoreType.DMA,
            pltpu.VMEM_SHARED((Np, 128), jnp.float32),
        ],
    )
    def k(g_hbm, src_hbm, dst_hbm, out_hbm, sbuf, dbuf, rowbuf, sem,
          shR):
        c = lax.axis_index("c")
        s = lax.axis_index("s")
        _zero_rows(rowbuf)
        for j in range(RT // CH):
            pltpu.sync_copy(rowbuf, shR.at[pl.ds(s * RT + j * CH, CH), :])
        plsc.subcore_barrier()
        ebase = c * (E // NC) + s * EPT

        def step(i, carry):
            off = ebase + i * CH
            pltpu.sync_copy(src_hbm.at[pl.ds(off, CH)], sbuf)
            pltpu.async_copy(g_hbm.at[sbuf], rowbuf, sem).wait()
            pltpu.sync_copy(dst_hbm.at[pl.ds(off, CH)], dbuf)
            pltpu.sync_copy(rowbuf, shR.at[dbuf], add=True)
            return carry

        lax.fori_loop(0, NCH, step, 0)
        plsc.subcore_barrier()
        for j in range(RT // CH):
            pltpu.sync_copy(shR.at[pl.ds(s * RT + j * CH, CH), :], rowbuf)
            pltpu.sync_copy(rowbuf,
                            out_hbm.at[c, pl.ds(s * RT + j * CH, CH), :])

    return k(g, src, dst)


def _sc_gat_cls(als, ald, c16, src, dst, emask, Np):
    ZROW = Np
    RT = Np // NS

    @functools.partial(
        pl.kernel,
        out_type=(_f32(NC * Np), _f32(NC * Np), _i32(E), _i32(E)),
        mesh=_mesh(),
        compiler_params=pltpu.CompilerParams(needs_layout_passes=False),
        scratch_types=[
            pltpu.VMEM((Np,), jnp.float32),
            pltpu.VMEM((Np,), jnp.float32),
            pltpu.VMEM((16,), jnp.float32),
            pltpu.VMEM((CH,), jnp.int32),
            pltpu.VMEM((CH,), jnp.int32),
            pltpu.VMEM((CH,), jnp.float32),
            pltpu.VMEM((CH,), jnp.int32),
            pltpu.VMEM((CH,), jnp.int32),
            pltpu.VMEM((CH,), jnp.float32),
            pltpu.VMEM((CH,), jnp.float32),
            pltpu.VMEM_SHARED((Np,), jnp.float32),
            pltpu.VMEM_SHARED((Np,), jnp.float32),
        ],
    )
    def k(als_hbm, ald_hbm, c_hbm, src_hbm, dst_hbm, em_hbm,
          denp_hbm, denm_hbm, sp_hbm, sm_hbm,
          alsb, aldb, cb, sbuf, dbuf, mbuf, spb, smb, enp, enm, shdp, shdm):
        c_ = lax.axis_index("c")
        s = lax.axis_index("s")
        for g in range(CH // 16):
            enp[pl.ds(g * 16, 16)] = jnp.zeros((16,), jnp.float32)
        for j in range(RT // CH):
            o = s * RT + j * CH
            pltpu.sync_copy(enp, shdp.at[pl.ds(o, CH)])
            pltpu.sync_copy(enp, shdm.at[pl.ds(o, CH)])
        pltpu.sync_copy(als_hbm, alsb)
        pltpu.sync_copy(ald_hbm, aldb)
        pltpu.sync_copy(c_hbm, cb)
        plsc.subcore_barrier()
        kvec = cb[pl.ds(0, 16)]
        Ms = kvec[1]
        ct = kvec[2]
        ebase = c_ * (E // NC) + s * EPT

        def step(i, carry):
            off = ebase + i * CH
            pltpu.sync_copy(src_hbm.at[pl.ds(off, CH)], sbuf)
            pltpu.sync_copy(dst_hbm.at[pl.ds(off, CH)], dbuf)
            pltpu.sync_copy(em_hbm.at[pl.ds(off, CH)], mbuf)
            for g in range(CH // 16):
                sl = pl.ds(g * 16, 16)
                si = sbuf[sl]
                di = dbuf[sl]
                av = plsc.load_gather(alsb, [si])
                dv = plsc.load_gather(aldb, [di])
                t = av + dv
                live = mbuf[sl] > 0.0
                kp = (t >= 0.0) & live
                km = (t < 0.0) & live
                zrows = ZROW + g * 16 + lax.iota(jnp.int32, 16)
                spb[sl] = jnp.where(kp, si, zrows)
                smb[sl] = jnp.where(km, si, zrows)
                enp[sl] = jnp.where(kp, jnp.exp(av - Ms), 0.0)
                enm[sl] = jnp.where(km, jnp.exp(0.2 * av - ct), 0.0)
            pltpu.sync_copy(enp, shdp.at[dbuf], add=True)
            pltpu.sync_copy(enm, shdm.at[dbuf], add=True)
            pltpu.sync_copy(spb, sp_hbm.at[pl.ds(off, CH)])
            pltpu.sync_copy(smb, sm_hbm.at[pl.ds(off, CH)])
            return carry

        lax.fori_loop(0, NCH, step, 0)
        plsc.subcore_barrier()
        for j in range(RT // CH):
            o = s * RT + j * CH
            pltpu.sync_copy(shdp.at[pl.ds(o, CH)], enp)
            pltpu.sync_copy(enp, denp_hbm.at[pl.ds(c_ * Np + o, CH)])
            pltpu.sync_copy(shdm.at[pl.ds(o, CH)], enm)
            pltpu.sync_copy(enm, denm_hbm.at[pl.ds(c_ * Np + o, CH)])

    dp, dm, sp, sm = k(als, ald, c16, src, dst, emask)
    return dp.reshape(NC, Np), dm.reshape(NC, Np), sp, sm


def _sc_gat_rows(u, v, sp, sm, dst, Np):
    RT = Np // NS

    @functools.partial(
        pl.kernel,
        out_type=(_f32(NC, Np, 128), _f32(NC, Np, 128)),
        mesh=_mesh(),
        compiler_params=pltpu.CompilerParams(needs_layout_passes=False),
        scratch_types=[
            pltpu.VMEM((CH,), jnp.int32),
            pltpu.VMEM((CH,), jnp.int32),
            pltpu.VMEM((CH,), jnp.int32),
            pltpu.VMEM((CH, 128), jnp.float32),
            pltpu.VMEM((CH, 128), jnp.float32),
            pltpu.SemaphoreType.DMA,
            pltpu.SemaphoreType.DMA,
            pltpu.VMEM_SHARED((Np, 128), jnp.float32),
            pltpu.VMEM_SHARED((Np, 128), jnp.float32),
        ],
    )
    def k(u_hbm, v_hbm, sp_hbm, sm_hbm, dst_hbm, Rp_hbm, Rm_hbm,
          spb, smb, dbuf, rowp, rowm, sem1, sem2, shRp, shRm):
        c_ = lax.axis_index("c")
        s = lax.axis_index("s")
        _zero_rows(rowp)
        for j in range(RT // CH):
            o = s * RT + j * CH
            pltpu.sync_copy(rowp, shRp.at[pl.ds(o, CH), :])
            pltpu.sync_copy(rowp, shRm.at[pl.ds(o, CH), :])
        plsc.subcore_barrier()
        ebase = c_ * (E // NC) + s * EPT

        def step(i, carry):
            off = ebase + i * CH
            pltpu.sync_copy(sp_hbm.at[pl.ds(off, CH)], spb)
            gp = pltpu.async_copy(u_hbm.at[spb], rowp, sem1)
            pltpu.sync_copy(sm_hbm.at[pl.ds(off, CH)], smb)
            gm = pltpu.async_copy(v_hbm.at[smb], rowm, sem2)
            pltpu.sync_copy(dst_hbm.at[pl.ds(off, CH)], dbuf)
            gp.wait()
            pltpu.sync_copy(rowp, shRp.at[dbuf], add=True)
            gm.wait()
            pltpu.sync_copy(rowm, shRm.at[dbuf], add=True)
            return carry

        lax.fori_loop(0, NCH, step, 0)
        plsc.subcore_barrier()
        for j in range(RT // CH):
            o = s * RT + j * CH
            pltpu.sync_copy(shRp.at[pl.ds(o, CH), :], rowp)
            pltpu.sync_copy(rowp, Rp_hbm.at[c_, pl.ds(o, CH), :])
            pltpu.sync_copy(shRm.at[pl.ds(o, CH), :], rowm)
            pltpu.sync_copy(rowm, Rm_hbm.at[c_, pl.ds(o, CH), :])

    return k(u, v, sp, sm, dst)


def _sc_gat(u, v, als, ald, c16, src, dst, emask):
    Np = als.shape[0]
    dp, dm, sp, sm = _sc_gat_cls(als, ald, c16, src, dst, emask, Np)
    Rp, Rm = _sc_gat_rows(u, v, sp, sm, dst, Np)
    return dp, dm, Rp, Rm


def _sc_pool(xs, rank, kk, npad_next=0, edges=None):
    Np = xs.shape[0]
    RT32 = Np // NW
    RCH = RT32 // CH if RT32 >= CH else 0

    with_edges = edges is not None
    out_type = [_f32(Np, 128)]
    if with_edges:
        out_type += [_i32(E), _i32(E), _f32(E)]

    scratch = [
        pltpu.VMEM((CH,), jnp.int32),
        pltpu.VMEM((CH, 128), jnp.float32),
        pltpu.VMEM((Np,), jnp.int32),
        pltpu.VMEM((CH,), jnp.int32),
        pltpu.VMEM((CH,), jnp.float32),
    ]

    def body(*refs):
        if with_edges:
            (xs_hbm, rk_hbm, src_hbm, dst_hbm, em_hbm,
             out_hbm, ns_hbm, nd_hbm, nm_hbm,
             rbuf, rowbuf, rktab, dbuf, mbuf) = refs
        else:
            (xs_hbm, rk_hbm, out_hbm,
             rbuf, rowbuf, rktab, dbuf, mbuf) = refs
        c = lax.axis_index("c")
        s = lax.axis_index("s")
        wid = c * NS + s
        rbase = wid * RT32

        if RCH:
            def rstep(i, carry):
                off = rbase + i * CH
                pltpu.sync_copy(xs_hbm.at[pl.ds(off, CH), :], rowbuf)
                pltpu.sync_copy(rk_hbm.at[pl.ds(off, CH)], rbuf)
                pltpu.sync_copy(rowbuf, out_hbm.at[rbuf])
                return carry
            lax.fori_loop(0, RCH, rstep, 0)

        if with_edges:
            pltpu.sync_copy(rk_hbm, rktab)
            ebase = c * (E // NC) + s * EPT
            kv = jnp.int32(kk)
            GPAD = jnp.int32(npad_next - kk)

            def estep(i, carry):
                off = ebase + i * CH
                pltpu.sync_copy(src_hbm.at[pl.ds(off, CH)], rbuf)
                pltpu.sync_copy(dst_hbm.at[pl.ds(off, CH)], dbuf)
                pltpu.sync_copy(em_hbm.at[pl.ds(off, CH)], mbuf)
                for g in range(CH // 16):
                    si = rbuf[pl.ds(g * 16, 16)]
                    di = dbuf[pl.ds(g * 16, 16)]
                    rs = plsc.load_gather(rktab, [si])
                    rd = plsc.load_gather(rktab, [di])
                    ks = rs < kv
                    kd = rd < kv
                    rbuf[pl.ds(g * 16, 16)] = jnp.where(ks, rs, 0)
                    dbuf[pl.ds(g * 16, 16)] = jnp.where(
                        kd, rd, kv + jnp.remainder(rd, GPAD))
                    mbuf[pl.ds(g * 16, 16)] = jnp.where(
                        ks & kd, mbuf[pl.ds(g * 16, 16)], 0.0)
                pltpu.sync_copy(rbuf, ns_hbm.at[pl.ds(off, CH)])
                pltpu.sync_copy(dbuf, nd_hbm.at[pl.ds(off, CH)])
                pltpu.sync_copy(mbuf, nm_hbm.at[pl.ds(off, CH)])
                return carry

            lax.fori_loop(0, NCH, estep, 0)

    kfun = functools.partial(
        pl.kernel, out_type=tuple(out_type), mesh=_mesh(),
        compiler_params=pltpu.CompilerParams(needs_layout_passes=False),
        scratch_types=scratch)(body)
    if with_edges:
        return kfun(xs, rank, *edges)
    return kfun(xs, rank)


def _pad_rows(a, np_):
    return jnp.pad(a, ((0, np_ - a.shape[0]),) + ((0, 0),) * (a.ndim - 1))


def kernel(x, edge_index, W_enc0, b_enc0, p0, W_gat1, b_gat1, a_src1, a_dst1,
           p1, W_gat2, b_gat2, a_src2, a_dst2, p2, W_mu, b_mu, W_lv, b_lv,
           W_lat, b_lat, W_d2, b_d2, W_d1, b_d1, W_d0, b_d0):
    src = edge_index[0]
    dst = edge_index[1]
    N0, Np0 = 10000, PADS[10000]
    k0, k1, k2 = 5000, 2500, 1250

    x_pad = jnp.pad(x, ((0, Np0 - N0), (0, 5)))
    W8 = jnp.pad(W_enc0, ((0, 5), (0, 0)))
    h0 = _tc_matmul(x_pad, W8)
    degparts = _sc_deg(dst, Np0)
    g, dsq, ideg = _tc_gcn_fin(degparts.T, h0)
    Sparts = _sc_scatter_rows(g, src, dst)
    xs0, s0 = _tc_gcn_combine(Sparts[0], Sparts[1], h0, dsq, ideg,
                              b_enc0.reshape(1, 128), p0.reshape(128, 1), N0)
    rank0 = _tc_rank(s0, s0.reshape(1, Np0))
    em0 = jnp.ones((E,), jnp.float32)
    sorted0, src1, dst1, em1 = _sc_pool(
        xs0, rank0.reshape(Np0), k0, PADS[k0], (src, dst, em0))

    def gat_level(x_lv, W, a_s, a_d, b, p, s_e, d_e, m_e, n_valid):
        Np = x_lv.shape[0]
        h, als, ald, exs, u, vtab, consts = _tc_gat_pre(
            x_lv, W, a_s.reshape(128, 1), a_d.reshape(128, 1))
        c16 = consts[0, :16].reshape(16)
        u_pad = jnp.pad(u, ((0, CH), (0, 0)))
        v_pad = jnp.pad(vtab, ((0, CH), (0, 0)))
        dp, dm, Rp, Rm = _sc_gat(u_pad, v_pad, als.reshape(Np),
                                 ald.reshape(Np), c16, s_e, d_e, m_e)
        xs, s = _tc_gat_post(Rp[0], Rp[1], Rm[0], Rm[1], h, exs, ald,
                             dp.T, dm.T, consts, b.reshape(1, 128),
                             p.reshape(128, 1), n_valid)
        rank = _tc_rank(s, s.reshape(1, Np))
        return xs, rank.reshape(Np)

    x1 = sorted0[:PADS[k0]]
    xs1, rank1 = gat_level(x1, W_gat1, a_src1, a_dst1, b_gat1, p1,
                           src1, dst1, em1, k0)
    sorted1, src2, dst2, em2 = _sc_pool(xs1, rank1, k1, PADS[k1],
                                        (src1, dst1, em1))

    x2 = sorted1[:PADS[k1]]
    xs2, rank2 = gat_level(x2, W_gat2, a_src2, a_dst2, b_gat2, p2,
                           src2, dst2, em2, k1)
    (sorted2,) = _sc_pool(xs2, rank2, k2)

    h3 = sorted2[:PADS[k2]]
    eps = jax.random.normal(jax.random.key(42), (k2, 32), dtype=jnp.float32)
    eps = _pad_rows(eps, PADS[k2])
    pad33 = lambda w: jnp.pad(w, ((0, 125), (0, 125)))
    padb = lambda b: jnp.pad(b, (0, 125)).reshape(1, 128)
    W_lat_p = jnp.pad(W_lat, ((0, 0), (0, 125)))
    b_lat_p = padb(b_lat)
    z, mu, lv = _tc_head(
        h3, eps, W_mu, b_mu.reshape(1, 32), W_lv, b_lv.reshape(1, 32),
        W_lat_p, b_lat_p, pad33(W_d2), padb(b_d2), pad33(W_d1), padb(b_d1),
        pad33(W_d0), padb(b_d0))
    return (z[:k2, :3], mu[:k2], lv[:k2])
